# Initial kernel scaffold; baseline (speedup 1.0000x reference)
import jax
import jax.numpy as jnp
from jax.experimental import pallas as pl


def kernel(x, W_qkv, b_qkv, W_proj, b_proj):
    raise NotImplementedError("write your pallas kernel here")


# speedup vs baseline: 1.7870x; 1.7870x over previous
import functools

import jax
import jax.numpy as jnp
from jax.experimental import pallas as pl
from jax.experimental.pallas import tpu as pltpu

DIM = 768
NUM_HEADS = 12
HEAD_DIM = DIM // NUM_HEADS
NUM_CLUSTERS = 4
CPAD = 8
NUM_SAMPLES = 32
NUM_ITERS = 10
SCALE = HEAD_DIM ** (-0.5)


def _kproj_body(x_ref, w_ref, b_ref, o_ref):
    kblk = jax.lax.dot_general(
        x_ref[0], w_ref[...], (((1,), (0,)), ((), ())),
        preferred_element_type=jnp.float32) + b_ref[...]
    nb = kblk.shape[0]
    o_ref[...] = kblk.reshape(nb, NUM_HEADS, HEAD_DIM).transpose(1, 0, 2)


def _kproj(x, wkT, bk, nb=512):
    B, N, C = x.shape
    return pl.pallas_call(
        _kproj_body,
        grid=(B, N // nb),
        in_specs=[
            pl.BlockSpec((1, nb, DIM), lambda b, i: (b, i, 0)),
            pl.BlockSpec((DIM, DIM), lambda b, i: (0, 0)),
            pl.BlockSpec((1, DIM), lambda b, i: (0, 0)),
        ],
        out_specs=pl.BlockSpec((NUM_HEADS, nb, HEAD_DIM), lambda b, i: (b, i, 0)),
        out_shape=jax.ShapeDtypeStruct((B * NUM_HEADS, N, HEAD_DIM), jnp.float32),
    )(x, wkT, bk)


def _kmeans_body(k_ref, c0_ref, g_ref, sidx_ref, ks_ref):
    kb = k_ref[0]
    N = kb.shape[0]
    ones_row = jnp.ones((1, HEAD_DIM), jnp.float32)
    k2 = jax.lax.dot_general(ones_row, kb * kb, (((1,), (1,)), ((), ())),
                             preferred_element_type=jnp.float32)
    row_iota = jax.lax.broadcasted_iota(jnp.int32, (CPAD, N), 0)
    lane_iota = jax.lax.broadcasted_iota(jnp.int32, (1, N), 1)
    valid = row_iota < NUM_CLUSTERS

    cent = c0_ref[0]

    def d2_of(c):
        ab = jax.lax.dot_general(c, kb, (((1,), (1,)), ((), ())),
                                 preferred_element_type=jnp.float32)
        c2 = jnp.sum(c * c, axis=1, keepdims=True)
        d2 = jnp.clip((k2 + c2) - 2.0 * ab, 0.0, None)
        return jnp.where(valid, d2, jnp.float32(1e30))

    for _ in range(NUM_ITERS):
        d2 = d2_of(cent)
        dmin = jnp.min(d2, axis=0, keepdims=True)
        cand = jnp.where(d2 == dmin, row_iota, CPAD)
        amin = jnp.min(cand, axis=0, keepdims=True)
        ohf = (row_iota == amin).astype(jnp.float32)
        sums = jax.lax.dot_general(ohf, kb, (((1,), (0,)), ((), ())),
                                   preferred_element_type=jnp.float32)
        counts = jnp.sum(ohf, axis=1, keepdims=True)
        newc = sums / (counts + 1e-06)
        cent = jnp.where(counts < 1e-06, cent, newc)

    d2 = d2_of(cent)
    dmin = jnp.min(d2, axis=0, keepdims=True)
    key_dists = jnp.sqrt(dmin + 1e-12)
    probs = key_dists / (jnp.sum(key_dists) + 1e-06)
    pert = jnp.log(probs + 1e-20) + g_ref[0]

    s_iota = jax.lax.broadcasted_iota(jnp.int32, (1, NUM_SAMPLES), 1)
    idx_row = jnp.zeros((1, NUM_SAMPLES), jnp.int32)
    for t in range(NUM_SAMPLES):
        m = jnp.max(pert)
        c = jnp.where(pert == m, lane_iota, N)
        idx = jnp.min(c)
        idx_row = jnp.where(s_iota == t, idx, idx_row)
        ks_ref[0, pl.ds(t, 1), :] = k_ref[0, pl.ds(idx, 1), :]
        pert = jnp.where(lane_iota == idx, jnp.float32(-1e30), pert)
    sidx_ref[0] = idx_row


def _kmeans(k3, cents0, g3, B, N):
    BH = B * NUM_HEADS
    return pl.pallas_call(
        _kmeans_body,
        grid=(BH,),
        in_specs=[
            pl.BlockSpec((1, N, HEAD_DIM), lambda i: (i, 0, 0)),
            pl.BlockSpec((1, CPAD, HEAD_DIM), lambda i: (i, 0, 0)),
            pl.BlockSpec((1, 1, N), lambda i: (i, 0, 0)),
        ],
        out_specs=[
            pl.BlockSpec((1, 1, NUM_SAMPLES), lambda i: (i, 0, 0)),
            pl.BlockSpec((1, NUM_SAMPLES, HEAD_DIM), lambda i: (i, 0, 0)),
        ],
        out_shape=[
            jax.ShapeDtypeStruct((BH, 1, NUM_SAMPLES), jnp.int32),
            jax.ShapeDtypeStruct((BH, NUM_SAMPLES, HEAD_DIM), jnp.float32),
        ],
    )(k3, cents0, g3)


def _vsample_body(xg_ref, w_ref, b_ref, o_ref):
    o_ref[0] = jax.lax.dot_general(
        xg_ref[0], w_ref[...], (((1,), (1,)), ((), ())),
        preferred_element_type=jnp.float32) + b_ref[0]


def _vsample(xg, W_qkv, bv, B):
    BH = B * NUM_HEADS
    return pl.pallas_call(
        _vsample_body,
        grid=(BH,),
        in_specs=[
            pl.BlockSpec((1, NUM_SAMPLES, DIM), lambda i: (i, 0, 0)),
            pl.BlockSpec((HEAD_DIM, DIM),
                         lambda i: (2 * NUM_HEADS + (i % NUM_HEADS), 0)),
            pl.BlockSpec((1, 1, HEAD_DIM), lambda i: (i % NUM_HEADS, 0, 0)),
        ],
        out_specs=pl.BlockSpec((1, NUM_SAMPLES, HEAD_DIM), lambda i: (i, 0, 0)),
        out_shape=jax.ShapeDtypeStruct((BH, NUM_SAMPLES, HEAD_DIM), jnp.float32),
    )(xg, W_qkv, bv)


def _attn_body(x_ref, wq_ref, bq_ref, ks_ref, vs_ref, wp_ref, bp_ref, o_ref):
    xb = x_ref[0]
    q = jax.lax.dot_general(xb, wq_ref[...], (((1,), (0,)), ((), ())),
                            preferred_element_type=jnp.float32) + bq_ref[...]
    acc = bp_ref[...] * jnp.ones((q.shape[0], 1), jnp.float32)
    for h in range(NUM_HEADS):
        qh = q[:, h * HEAD_DIM:(h + 1) * HEAD_DIM]
        ksh = ks_ref[0, h * NUM_SAMPLES:(h + 1) * NUM_SAMPLES, :]
        vsh = vs_ref[0, h * NUM_SAMPLES:(h + 1) * NUM_SAMPLES, :]
        logits = jax.lax.dot_general(qh, ksh, (((1,), (1,)), ((), ())),
                                     preferred_element_type=jnp.float32) * SCALE
        m = jnp.max(logits, axis=1, keepdims=True)
        e = jnp.exp(logits - m)
        a = e / jnp.sum(e, axis=1, keepdims=True)
        oh = jax.lax.dot_general(a, vsh, (((1,), (0,)), ((), ())),
                                 preferred_element_type=jnp.float32)
        wph = wp_ref[h * HEAD_DIM:(h + 1) * HEAD_DIM, :]
        acc = acc + jax.lax.dot_general(oh, wph, (((1,), (0,)), ((), ())),
                                        preferred_element_type=jnp.float32)
    o_ref[0] = acc


def _attn(x, wqT, bq, ks, vs, wpT, bp, nb=512):
    B, N, _ = x.shape
    return pl.pallas_call(
        _attn_body,
        grid=(B, N // nb),
        in_specs=[
            pl.BlockSpec((1, nb, DIM), lambda b, i: (b, i, 0)),
            pl.BlockSpec((DIM, DIM), lambda b, i: (0, 0)),
            pl.BlockSpec((1, DIM), lambda b, i: (0, 0)),
            pl.BlockSpec((1, NUM_HEADS * NUM_SAMPLES, HEAD_DIM),
                         lambda b, i: (b, 0, 0)),
            pl.BlockSpec((1, NUM_HEADS * NUM_SAMPLES, HEAD_DIM),
                         lambda b, i: (b, 0, 0)),
            pl.BlockSpec((DIM, DIM), lambda b, i: (0, 0)),
            pl.BlockSpec((1, DIM), lambda b, i: (0, 0)),
        ],
        out_specs=pl.BlockSpec((1, nb, DIM), lambda b, i: (b, i, 0)),
        out_shape=jax.ShapeDtypeStruct((B, N, DIM), jnp.float32),
    )(x, wqT, bq, ks, vs, wpT, bp)


def kernel(x, W_qkv, b_qkv, W_proj, b_proj):
    B, N, C = x.shape
    H, hd, S = NUM_HEADS, HEAD_DIM, NUM_SAMPLES
    BH = B * H

    rng = jax.random.key(42)
    k1, k2 = jax.random.split(rng)
    rand_idx = jax.random.randint(k1, (B, H, NUM_CLUSTERS), 0, N)
    u = jax.random.uniform(k2, (B, H, N))
    g = -jnp.log(-jnp.log(u + 1e-20) + 1e-20)
    g3 = g.reshape(BH, 1, N)

    wkT = W_qkv[C:2 * C, :].T
    bk = b_qkv[C:2 * C].reshape(1, C)
    k3 = _kproj(x, wkT, bk)

    cents0 = jnp.take_along_axis(
        k3.reshape(B, H, N, hd),
        rand_idx[..., None, None].repeat(hd, axis=-1).reshape(B, H, NUM_CLUSTERS, hd),
        axis=2)
    cents0 = jnp.pad(cents0, ((0, 0), (0, 0), (0, CPAD - NUM_CLUSTERS), (0, 0)))
    cents0 = cents0.reshape(BH, CPAD, hd)

    sidx, ks = _kmeans(k3, cents0, g3, B, N)
    sidx_bh = sidx.reshape(B, H, S)

    xg = jnp.take_along_axis(
        x[:, None, :, :],
        jnp.broadcast_to(sidx_bh[..., None], (B, H, S, C)),
        axis=2).reshape(BH, S, C)
    bv = b_qkv[2 * C:].reshape(H, 1, hd)
    vs = _vsample(xg, W_qkv, bv, B)

    wqT = W_qkv[:C, :].T
    bq = b_qkv[:C].reshape(1, C)
    wpT = W_proj.T
    bp = b_proj.reshape(1, C)
    out = _attn(x, wqT, bq,
                ks.reshape(B, H * S, hd), vs.reshape(B, H * S, hd),
                wpT, bp)
    return out


# speedup vs baseline: 2.3252x; 1.3012x over previous
import jax
import jax.numpy as jnp
from jax.experimental import pallas as pl
from jax.experimental.pallas import tpu as pltpu

DIM = 768
NUM_HEADS = 12
HEAD_DIM = DIM // NUM_HEADS
NUM_CLUSTERS = 4
CPAD = 8
NUM_SAMPLES = 32
NUM_ITERS = 10
SCALE = HEAD_DIM ** (-0.5)
NB = 512
N_SEQ = 8192
N_CHUNKS = N_SEQ // NB
SUB = 8
LANES = N_SEQ // SUB


def _dotg(a, b, dims, prec=None):
    return jax.lax.dot_general(a, b, (dims, ((), ())),
                               preferred_element_type=jnp.float32)


def _kmix_body(x_ref, wk_ref, bk_ref, ridx_ref, g_ref,
               sidx_ref, kbd_ref, k_scr, k2_scr):
    s = pl.program_id(1)
    n_chunks = pl.num_programs(1) - 1
    n = k_scr.shape[0]
    lanes = n // SUB

    @pl.when(s < n_chunks)
    def _proj():
        blk = _dotg(x_ref[0], wk_ref[...], ((1,), (0,))) + bk_ref[...]
        k_scr[pl.ds(s * NB, NB), :] = blk

    @pl.when(s == n_chunks)
    def _kmeans():
        kbd_ref[0] = jnp.zeros((DIM, NUM_HEADS * NUM_SAMPLES), jnp.float32)
        row_iota = jax.lax.broadcasted_iota(jnp.int32, (CPAD, n), 0)
        valid = row_iota < NUM_CLUSTERS
        flat_iota = (jax.lax.broadcasted_iota(jnp.int32, (SUB, lanes), 0) * lanes
                     + jax.lax.broadcasted_iota(jnp.int32, (SUB, lanes), 1))
        s_iota = jax.lax.broadcasted_iota(jnp.int32, (1, NUM_SAMPLES), 1)
        c_iota = jax.lax.broadcasted_iota(jnp.int32, (NUM_SAMPLES, 1), 0)
        samp_lane = jax.lax.broadcasted_iota(
            jnp.int32, (NUM_SAMPLES, n), 1)

        ones_row = jnp.ones((1, HEAD_DIM), jnp.float32)

        def kb_of(h):
            return k_scr[:, h * HEAD_DIM:(h + 1) * HEAD_DIM]

        for h in range(NUM_HEADS):
            kb = kb_of(h)
            k2_scr[pl.ds(h, 1), :] = _dotg(ones_row, kb * kb, ((1,), (1,)))

        def d2_of(cent, h):
            ab = _dotg(cent, kb_of(h), ((1,), (1,)))
            c2 = jnp.sum(cent * cent, axis=1, keepdims=True)
            d2 = jnp.clip((k2_scr[pl.ds(h, 1), :] + c2) - 2.0 * ab, 0.0, None)
            return jnp.where(valid, d2, jnp.float32(1e30))

        def km_iter(_, cents):
            new = []
            for h in range(NUM_HEADS):
                d2 = d2_of(cents[h], h)
                dmin = jnp.min(d2, axis=0, keepdims=True)
                cand = jnp.where(d2 == dmin, row_iota, CPAD)
                amin = jnp.min(cand, axis=0, keepdims=True)
                ohf = (row_iota == amin).astype(jnp.float32)
                sums = _dotg(ohf, kb_of(h), ((1,), (0,)))
                counts = jnp.sum(ohf, axis=1, keepdims=True)
                newc = sums / (counts + 1e-06)
                new.append(jnp.where(counts < 1e-06, cents[h], newc))
            return tuple(new)

        cents = []
        for h in range(NUM_HEADS):
            rows = [k_scr[pl.ds(ridx_ref[0, h, c], 1),
                          :][:, h * HEAD_DIM:(h + 1) * HEAD_DIM]
                    for c in range(NUM_CLUSTERS)]
            rows.append(jnp.zeros((CPAD - NUM_CLUSTERS, HEAD_DIM),
                                  jnp.float32))
            cents.append(jnp.concatenate(rows, axis=0))
        cents = jax.lax.fori_loop(0, NUM_ITERS, km_iter, tuple(cents))

        for h in range(NUM_HEADS):
            d2 = d2_of(cents[h], h)
            dmin = jnp.min(d2, axis=0, keepdims=True)
            key_dists = jnp.sqrt(dmin + 1e-12)
            probs = key_dists / (jnp.sum(key_dists) + 1e-06)
            pert = jnp.log(probs + 1e-20).reshape(SUB, lanes) + g_ref[0, h]

            def pick(t, carry):
                p, idx_row, idx_col = carry
                m = jnp.max(p)
                cand = jnp.where(p == m, flat_iota, n)
                idx = jnp.min(cand)
                idx_row = jnp.where(s_iota == t, idx, idx_row)
                idx_col = jnp.where(c_iota == t, idx, idx_col)
                p = jnp.where(flat_iota == idx, jnp.float32(-1e30), p)
                return p, idx_row, idx_col

            idx_row = jnp.zeros((1, NUM_SAMPLES), jnp.int32)
            idx_col = jnp.zeros((NUM_SAMPLES, 1), jnp.int32)
            _, idx_row, idx_col = jax.lax.fori_loop(
                0, NUM_SAMPLES, pick, (pert, idx_row, idx_col))
            sidx_ref[0, pl.ds(h, 1), :] = idx_row
            oh = (samp_lane == idx_col).astype(jnp.float32)
            ksT = _dotg(kb_of(h), oh, ((0,), (1,)))
            kbd_ref[0, pl.ds(h * HEAD_DIM, HEAD_DIM),
                    pl.ds(h * NUM_SAMPLES, NUM_SAMPLES)] = ksT


def _kmix(x, wkT, bk, rand_idx, g4):
    B, N, C = x.shape
    n_chunks = N // NB
    return pl.pallas_call(
        _kmix_body,
        grid=(B, n_chunks + 1),
        in_specs=[
            pl.BlockSpec((1, NB, DIM),
                         lambda b, s: (b, jnp.minimum(s, n_chunks - 1), 0)),
            pl.BlockSpec((DIM, DIM), lambda b, s: (0, 0)),
            pl.BlockSpec((1, DIM), lambda b, s: (0, 0)),
            pl.BlockSpec((1, NUM_HEADS, NUM_CLUSTERS),
                         lambda b, s: (b, 0, 0)),
            pl.BlockSpec((1, NUM_HEADS, SUB, N // SUB),
                         lambda b, s: (b, 0, 0, 0)),
        ],
        out_specs=[
            pl.BlockSpec((1, NUM_HEADS, NUM_SAMPLES), lambda b, s: (b, 0, 0)),
            pl.BlockSpec((1, DIM, NUM_HEADS * NUM_SAMPLES),
                         lambda b, s: (b, 0, 0)),
        ],
        out_shape=[
            jax.ShapeDtypeStruct((B, NUM_HEADS, NUM_SAMPLES), jnp.int32),
            jax.ShapeDtypeStruct((B, DIM, NUM_HEADS * NUM_SAMPLES),
                                 jnp.float32),
        ],
        scratch_shapes=[pltpu.VMEM((N, DIM), jnp.float32),
                        pltpu.VMEM((NUM_HEADS, N), jnp.float32)],
    )(x, wkT, bk, rand_idx, g4)


def _pmat_body(xg_ref, wv_ref, bv_ref, wp_ref, o_ref):
    vs = _dotg(xg_ref[0], wv_ref[...], ((1,), (1,))) + bv_ref[0]
    o_ref[0] = _dotg(vs, wp_ref[...], ((1,), (0,)))


def _pmat(xg, W_qkv, bv, wpT, B):
    BH = B * NUM_HEADS
    return pl.pallas_call(
        _pmat_body,
        grid=(BH,),
        in_specs=[
            pl.BlockSpec((1, NUM_SAMPLES, DIM), lambda i: (i, 0, 0)),
            pl.BlockSpec((HEAD_DIM, DIM),
                         lambda i: (2 * NUM_HEADS + (i % NUM_HEADS), 0)),
            pl.BlockSpec((1, 1, HEAD_DIM), lambda i: (i % NUM_HEADS, 0, 0)),
            pl.BlockSpec((HEAD_DIM, DIM), lambda i: (i % NUM_HEADS, 0)),
        ],
        out_specs=pl.BlockSpec((1, NUM_SAMPLES, DIM),
                               lambda i: (i // NUM_HEADS, i % NUM_HEADS, 0)),
        out_shape=jax.ShapeDtypeStruct((B, NUM_HEADS * NUM_SAMPLES, DIM),
                                       jnp.float32),
    )(xg, W_qkv, bv, wpT)


def _attn_body(x_ref, wq_ref, bq_ref, kbd_ref, ones_ref, p_ref, bp_ref,
               o_ref):
    xb = x_ref[0]
    q = _dotg(xb, wq_ref[...], ((1,), (0,))) + bq_ref[...]
    logits = _dotg(q, kbd_ref[0], ((1,), (0,))) * SCALE
    e = jnp.exp(logits)
    den = _dotg(e, ones_ref[...], ((1,), (0,)))
    a = e / den
    o_ref[0] = _dotg(a, p_ref[0], ((1,), (0,))) + bp_ref[...]


def _attn(x, wqT, bq, kbd, bdones, P, bp, nb=512):
    B, N, _ = x.shape
    HS = NUM_HEADS * NUM_SAMPLES
    return pl.pallas_call(
        _attn_body,
        grid=(B, N // nb),
        in_specs=[
            pl.BlockSpec((1, nb, DIM), lambda b, i: (b, i, 0)),
            pl.BlockSpec((DIM, DIM), lambda b, i: (0, 0)),
            pl.BlockSpec((1, DIM), lambda b, i: (0, 0)),
            pl.BlockSpec((1, DIM, HS), lambda b, i: (b, 0, 0)),
            pl.BlockSpec((HS, HS), lambda b, i: (0, 0)),
            pl.BlockSpec((1, HS, DIM), lambda b, i: (b, 0, 0)),
            pl.BlockSpec((1, DIM), lambda b, i: (0, 0)),
        ],
        out_specs=pl.BlockSpec((1, nb, DIM), lambda b, i: (b, i, 0)),
        out_shape=jax.ShapeDtypeStruct((B, N, DIM), jnp.float32),
    )(x, wqT, bq, kbd, bdones, P, bp)


def kernel(x, W_qkv, b_qkv, W_proj, b_proj):
    B, N, C = x.shape
    H, hd, S = NUM_HEADS, HEAD_DIM, NUM_SAMPLES

    rng = jax.random.key(42)
    r1, r2 = jax.random.split(rng)
    rand_idx = jax.random.randint(r1, (B, H, NUM_CLUSTERS), 0, N)
    u = jax.random.uniform(r2, (B, H, N))
    g4 = (-jnp.log(-jnp.log(u + 1e-20) + 1e-20)).reshape(B, H, SUB, N // SUB)

    wkT = W_qkv[C:2 * C, :].T
    bk = b_qkv[C:2 * C].reshape(1, C)

    sidx, kbd = _kmix(x, wkT, bk, rand_idx.astype(jnp.int32), g4)

    xg = jnp.take_along_axis(
        x[:, None, :, :],
        jnp.broadcast_to(sidx[..., None], (B, H, S, C)),
        axis=2).reshape(B * H, S, C)
    bv = b_qkv[2 * C:].reshape(H, 1, hd)
    wpT = W_proj.T
    P = _pmat(xg, W_qkv, bv, wpT, B)

    wqT = W_qkv[:C, :].T
    bq = b_qkv[:C].reshape(1, C)
    bp = b_proj.reshape(1, C)
    bdones = jnp.kron(jnp.eye(H, dtype=jnp.float32),
                      jnp.ones((S, S), jnp.float32))
    return _attn(x, wqT, bq, kbd, bdones, P, bp)
